# baseline (speedup 1.0000x reference)
import jax
import jax.numpy as jnp
from jax import lax
from jax.experimental import pallas as pl
from jax.experimental.pallas import tpu as pltpu
from jax.experimental.pallas import tpu_sc as plsc

_BATCH = 16384
_V = 1000000
_D = 64
_NC = 2
_NS = 16
_NW = _NC * _NS
_N = _BATCH // _NW
_CHUNK = 128
_NCHUNK = _N // _CHUNK
_EPS = 1e-6
_MARGIN = 1.0

_GDNUMS = lax.GatherDimensionNumbers(
    offset_dims=(), collapsed_slice_dims=(0,), start_index_map=(0,))


def _sqrt16(x):
    i = plsc.bitcast(x, jnp.int32)
    y = plsc.bitcast(jnp.int32(0x5F3759DF) - (i >> 1), jnp.float32)
    half_x = 0.5 * x
    for _ in range(4):
        y = y * (1.5 - half_x * y * y)
    return x * y


def _bcast(v, idx16):
    return lax.gather(v, idx16[:, None], _GDNUMS, slice_sizes=(1,),
                      mode=lax.GatherScatterMode.PROMISE_IN_BOUNDS)


def _allsum16(v, lane):
    for s in (8, 4, 2, 1):
        idx = (lane + s) & 15
        v = v + lax.gather(v, idx[:, None], _GDNUMS, slice_sizes=(1,),
                           mode=lax.GatherScatterMode.PROMISE_IN_BOUNDS)
    return v


def _body(ph, pr, pt, nh, nr, nt, ent, rel, out,
          ih, ir, it, xh, xr, xt, hb, rb, tb,
          ss_pos, ss_neg, part, sem_a, sem_b, sem_c):
    wid = lax.axis_index("s") * _NC + lax.axis_index("c")
    lane = lax.iota(jnp.int32, 16)

    def side(h_idx, r_idx, t_idx, ss_ref):
        pltpu.sync_copy(h_idx.at[wid], ih)
        pltpu.sync_copy(r_idx.at[wid], ir)
        pltpu.sync_copy(t_idx.at[wid], it)

        for j in range(_NCHUNK):
            for s in range(_CHUNK // 16):
                sl = pl.ds(s * 16, 16)
                xh[sl] = ih[j, sl] >> 1
                xr[sl] = ir[j, sl] >> 1
                xt[sl] = it[j, sl] >> 1
            ch = pltpu.async_copy(ent.at[xh], hb, sem_a)
            cr = pltpu.async_copy(rel.at[xr], rb, sem_b)
            ct = pltpu.async_copy(ent.at[xt], tb, sem_c)
            ch.wait()
            cr.wait()
            ct.wait()

            def group(g, carry):
                gs = pl.ds(g * 16, 16)
                par_h = ih[j, gs] & 1
                par_r = ir[j, gs] & 1
                par_t = it[j, gs] & 1
                acc = jnp.zeros((16,), jnp.float32)
                for i in range(16):
                    b = g * 16 + i
                    bi = (lane & 0) + i
                    mh = _bcast(par_h, bi) == 1
                    mr = _bcast(par_r, bi) == 1
                    mt = _bcast(par_t, bi) == 1
                    p = None
                    for k in range(_D // 16):
                        lo = pl.ds(k * 16, 16)
                        hi = pl.ds(_D + k * 16, 16)
                        hv = jnp.where(mh, hb[b, hi], hb[b, lo])
                        rv = jnp.where(mr, rb[b, hi], rb[b, lo])
                        tv = jnp.where(mt, tb[b, hi], tb[b, lo])
                        d = hv + rv - tv + _EPS
                        sq = d * d
                        p = sq if p is None else p + sq
                    s16 = _allsum16(p, lane)
                    acc = jnp.where(lane == i, s16, acc)
                ss_ref[j, gs] = acc
                return carry

            lax.fori_loop(0, _CHUNK // 16, group, 0)

    side(ph, pr, pt, ss_pos)
    side(nh, nr, nt, ss_neg)

    total = jnp.zeros((16,), jnp.float32)
    for j in range(_NCHUNK):
        for s in range(_CHUNK // 16):
            p = _sqrt16(ss_pos[j, pl.ds(s * 16, 16)])
            n = _sqrt16(ss_neg[j, pl.ds(s * 16, 16)])
            total = total + jnp.maximum(p - n + _MARGIN, 0.0)
    part[...] = total
    pltpu.sync_copy(part, out.at[wid])


@jax.jit
def kernel(pos_x, neg_x, ent_emb, rel_emb):
    mesh = plsc.VectorSubcoreMesh(core_axis_name="c", subcore_axis_name="s",
                                  num_cores=_NC, num_subcores=_NS)
    k = pl.kernel(
        _body,
        out_type=jax.ShapeDtypeStruct((_NW, 16), jnp.float32),
        mesh=mesh,
        scratch_types=[
            pltpu.VMEM((_NCHUNK, _CHUNK), jnp.int32),
            pltpu.VMEM((_NCHUNK, _CHUNK), jnp.int32),
            pltpu.VMEM((_NCHUNK, _CHUNK), jnp.int32),
            pltpu.VMEM((_CHUNK,), jnp.int32),
            pltpu.VMEM((_CHUNK,), jnp.int32),
            pltpu.VMEM((_CHUNK,), jnp.int32),
            pltpu.VMEM((_CHUNK, 2 * _D), jnp.float32),
            pltpu.VMEM((_CHUNK, 2 * _D), jnp.float32),
            pltpu.VMEM((_CHUNK, 2 * _D), jnp.float32),
            pltpu.VMEM((_NCHUNK, _CHUNK), jnp.float32),
            pltpu.VMEM((_NCHUNK, _CHUNK), jnp.float32),
            pltpu.VMEM((16,), jnp.float32),
            pltpu.SemaphoreType.DMA,
            pltpu.SemaphoreType.DMA,
            pltpu.SemaphoreType.DMA,
        ],
        compiler_params=pltpu.CompilerParams(
            needs_layout_passes=False, use_tc_tiling_on_sc=False),
    )

    def cols(x):
        return [x[:, i].reshape(_NW, _NCHUNK, _CHUNK) for i in range(3)]

    ph, pr, pt = cols(pos_x)
    nh, nr, nt = cols(neg_x)
    partials = k(ph, pr, pt, nh, nr, nt,
                 ent_emb.reshape(_V // 2, 2 * _D),
                 rel_emb.reshape(_V // 2, 2 * _D))
    return jnp.sum(partials) / jnp.float32(_BATCH)


# speedup vs baseline: 1.0028x; 1.0028x over previous
import jax
import jax.numpy as jnp
from jax import lax
from jax.experimental import pallas as pl
from jax.experimental.pallas import tpu as pltpu
from jax.experimental.pallas import tpu_sc as plsc

_BATCH = 16384
_V = 1000000
_D = 64
_NC = 2
_NS = 16
_NW = _NC * _NS
_N = _BATCH // _NW
_CHUNK = 128
_NCHUNK = _N // _CHUNK
_EPS = 1e-6
_MARGIN = 1.0

_GDNUMS = lax.GatherDimensionNumbers(
    offset_dims=(), collapsed_slice_dims=(0,), start_index_map=(0,))


def _sqrt16(x):
    i = plsc.bitcast(x, jnp.int32)
    y = plsc.bitcast(jnp.int32(0x5F3759DF) - (i >> 1), jnp.float32)
    half_x = 0.5 * x
    for _ in range(4):
        y = y * (1.5 - half_x * y * y)
    return x * y


def _bcast(v, idx16):
    return lax.gather(v, idx16[:, None], _GDNUMS, slice_sizes=(1,),
                      mode=lax.GatherScatterMode.PROMISE_IN_BOUNDS)


def _allsum16(v, lane):
    for s in (8, 4, 2, 1):
        idx = (lane + s) & 15
        v = v + lax.gather(v, idx[:, None], _GDNUMS, slice_sizes=(1,),
                           mode=lax.GatherScatterMode.PROMISE_IN_BOUNDS)
    return v


def _body(ph, pr, pt, nh, nr, nt, ent, rel, out,
          ih, ir, it, xh, xr, xt, hb, rb, tb,
          ss_pos, ss_neg, part, sem_a, sem_b, sem_c):
    wid = lax.axis_index("s") * _NC + lax.axis_index("c")
    lane = lax.iota(jnp.int32, 16)

    def side(h_idx, r_idx, t_idx, ss_ref):
        pltpu.sync_copy(h_idx.at[wid], ih)
        pltpu.sync_copy(r_idx.at[wid], ir)
        pltpu.sync_copy(t_idx.at[wid], it)

        for j in range(_NCHUNK):
            for s in range(_CHUNK // 16):
                sl = pl.ds(s * 16, 16)
                xh[sl] = ih[j, sl] >> 1
                xr[sl] = ir[j, sl] >> 1
                xt[sl] = it[j, sl] >> 1
            ch = pltpu.async_copy(ent.at[xh], hb, sem_a)
            cr = pltpu.async_copy(rel.at[xr], rb, sem_b)
            ct = pltpu.async_copy(ent.at[xt], tb, sem_c)
            ch.wait()
            cr.wait()
            ct.wait()

            def group(g, carry):
                gs = pl.ds(g * 16, 16)
                par_h = ih[j, gs] & 1
                par_r = ir[j, gs] & 1
                par_t = it[j, gs] & 1
                acc = jnp.zeros((16,), jnp.float32)
                for i in range(16):
                    b = g * 16 + i
                    bi = (lane & 0) + i
                    mh = _bcast(par_h, bi) == 1
                    mr = _bcast(par_r, bi) == 1
                    mt = _bcast(par_t, bi) == 1
                    p = None
                    for k in range(_D // 16):
                        lo = pl.ds(k * 16, 16)
                        hi = pl.ds(_D + k * 16, 16)
                        hv = jnp.where(mh, hb[b, hi], hb[b, lo])
                        rv = jnp.where(mr, rb[b, hi], rb[b, lo])
                        tv = jnp.where(mt, tb[b, hi], tb[b, lo])
                        d = hv + rv - tv + _EPS
                        sq = d * d
                        p = sq if p is None else p + sq
                    s16 = _allsum16(p, lane)
                    acc = jnp.where(lane == i, s16, acc)
                ss_ref[j, gs] = acc
                return carry

            lax.fori_loop(0, _CHUNK // 16, group, 0)

    side(ph, pr, pt, ss_pos)
    side(nh, nr, nt, ss_neg)

    total = jnp.zeros((16,), jnp.float32)
    for j in range(_NCHUNK):
        for s in range(_CHUNK // 16):
            p = _sqrt16(ss_pos[j, pl.ds(s * 16, 16)])
            n = _sqrt16(ss_neg[j, pl.ds(s * 16, 16)])
            total = total + jnp.maximum(p - n + _MARGIN, 0.0)
    for s in range(_CHUNK // 16):
        part[pl.ds(s * 16, 16)] = total if s == 0 else jnp.zeros(
            (16,), jnp.float32)
    pltpu.sync_copy(part, out.at[wid])


@jax.jit
def kernel(pos_x, neg_x, ent_emb, rel_emb):
    mesh = plsc.VectorSubcoreMesh(core_axis_name="c", subcore_axis_name="s",
                                  num_cores=_NC, num_subcores=_NS)
    k = pl.kernel(
        _body,
        out_type=jax.ShapeDtypeStruct((_NW, _CHUNK), jnp.float32),
        mesh=mesh,
        scratch_types=[
            pltpu.VMEM((_NCHUNK, _CHUNK), jnp.int32),
            pltpu.VMEM((_NCHUNK, _CHUNK), jnp.int32),
            pltpu.VMEM((_NCHUNK, _CHUNK), jnp.int32),
            pltpu.VMEM((_CHUNK,), jnp.int32),
            pltpu.VMEM((_CHUNK,), jnp.int32),
            pltpu.VMEM((_CHUNK,), jnp.int32),
            pltpu.VMEM((_CHUNK, 2 * _D), jnp.float32),
            pltpu.VMEM((_CHUNK, 2 * _D), jnp.float32),
            pltpu.VMEM((_CHUNK, 2 * _D), jnp.float32),
            pltpu.VMEM((_NCHUNK, _CHUNK), jnp.float32),
            pltpu.VMEM((_NCHUNK, _CHUNK), jnp.float32),
            pltpu.VMEM((_CHUNK,), jnp.float32),
            pltpu.SemaphoreType.DMA,
            pltpu.SemaphoreType.DMA,
            pltpu.SemaphoreType.DMA,
        ],
        compiler_params=pltpu.CompilerParams(
            needs_layout_passes=False, use_tc_tiling_on_sc=True),
    )

    def cols(x):
        return [x[:, i].reshape(_NW, _NCHUNK, _CHUNK) for i in range(3)]

    ph, pr, pt = cols(pos_x)
    nh, nr, nt = cols(neg_x)
    partials = k(ph, pr, pt, nh, nr, nt,
                 ent_emb.reshape(_V // 2, 2 * _D),
                 rel_emb.reshape(_V // 2, 2 * _D))
    return jnp.sum(partials[:, :16]) / jnp.float32(_BATCH)
